# baseline (speedup 1.0000x reference)
import functools

import jax
import jax.numpy as jnp
import numpy as np
from jax import lax
from jax.experimental import pallas as pl
from jax.experimental.pallas import tpu as pltpu
from jax.experimental.pallas import tpu_sc as plsc

IN_FEATURES = 2048
OUT_FEATURES = 2048
NUM_EXPERTS = 8
LORA_RANK = 16
SPARSEGEN_LAMBDA = 0.5
LORA_SCALING = 2.0

LANES = 128
NCAT = OUT_FEATURES + LANES
BT = 1024

_SC_INFO = plsc.get_sparse_core_info()
_NW = _SC_INFO.num_cores * _SC_INFO.num_subcores


def _logits_body(x_ref, rwt_ref, rb_ref, zt_ref):
    xb = x_ref[...].astype(jnp.bfloat16)
    z = jax.lax.dot_general(xb, rwt_ref[...], (((1,), (1,)), ((), ())),
                            preferred_element_type=jnp.float32)
    zt = z[:, :NUM_EXPERTS].T
    zt_ref[...] = (zt + rb_ref[...][:, :1]) * (1.0 / (1.0 - SPARSEGEN_LAMBDA))


def _sc_sparsemax_body(zt_hbm, wt_hbm, zvm, wvm):
    tpb = zt_hbm.shape[1] // _NW
    wid = lax.axis_index("s") * _SC_INFO.num_cores + lax.axis_index("c")
    base = wid * tpb
    pltpu.sync_copy(zt_hbm.at[:, pl.ds(base, tpb)], zvm)
    for j in range(tpb // 16):
        zs = [zvm[e, pl.ds(j * 16, 16)] for e in range(NUM_EXPERTS)]
        act = [jnp.full((16,), 1.0, jnp.float32)] * NUM_EXPERTS
        tau = jnp.zeros((16,), jnp.float32)
        for _ in range(NUM_EXPERTS):
            n = act[0]
            s = zs[0] * act[0]
            for e in range(1, NUM_EXPERTS):
                n = n + act[e]
                s = s + zs[e] * act[e]
            tau = (s - 1.0) / n
            act = [jnp.where(zs[e] > tau, 1.0, 0.0)
                   for e in range(NUM_EXPERTS)]
        for e in range(NUM_EXPERTS):
            wvm[e, pl.ds(j * 16, 16)] = jnp.maximum(zs[e] - tau, 0.0)
    pltpu.sync_copy(wvm, wt_hbm.at[:, pl.ds(base, tpb)])


def _sc_sparsemax(zt):
    E, T = zt.shape
    tpb = T // _NW
    mesh = plsc.VectorSubcoreMesh(core_axis_name="c", subcore_axis_name="s")
    return pl.kernel(
        _sc_sparsemax_body,
        mesh=mesh,
        out_type=jax.ShapeDtypeStruct((E, T), jnp.float32),
        scratch_types=[
            pltpu.VMEM((E, tpb), jnp.float32),
            pltpu.VMEM((E, tpb), jnp.float32),
        ],
    )(zt)


def _dense_body(x_ref, wcat_ref, b_ref, wt_ref, bcat_ref, rep_ref, o_ref):
    xb = x_ref[...].astype(jnp.bfloat16)
    big = jax.lax.dot_general(xb, wcat_ref[...], (((1,), (1,)), ((), ())),
                              preferred_element_type=jnp.float32)
    base = big[:, :OUT_FEATURES]
    h = big[:, OUT_FEATURES:]

    w = wt_ref[...].T
    wrep = jnp.dot(w, rep_ref[...], preferred_element_type=jnp.float32)
    eo = jnp.dot((wrep * h).astype(jnp.bfloat16), bcat_ref[...],
                 preferred_element_type=jnp.float32)

    o_ref[...] = base + b_ref[...] + LORA_SCALING * eo


@functools.partial(jax.jit, static_argnames=())
def kernel(x, W_base, b_base, router_W, router_b, lora_A, lora_B):
    T = x.shape[0]
    b2 = b_base.reshape(1, OUT_FEATURES)
    rwt = jnp.zeros((LANES, IN_FEATURES), jnp.float32).at[:NUM_EXPERTS].set(
        router_W.T).astype(jnp.bfloat16)
    rb = jnp.broadcast_to(router_b.reshape(NUM_EXPERTS, 1),
                          (NUM_EXPERTS, LANES))
    acat_t = lora_A.transpose(0, 2, 1).reshape(NUM_EXPERTS * LORA_RANK,
                                               IN_FEATURES)
    wcat = jnp.concatenate([W_base, acat_t], axis=0).astype(jnp.bfloat16)
    bcat = lora_B.reshape(NUM_EXPERTS * LORA_RANK,
                          OUT_FEATURES).astype(jnp.bfloat16)
    rep = np.zeros((NUM_EXPERTS, LANES), np.float32)
    for e in range(NUM_EXPERTS):
        rep[e, e * LORA_RANK:(e + 1) * LORA_RANK] = 1.0
    rep = jnp.asarray(rep)

    grid = (T // BT,)

    zt = pl.pallas_call(
        _logits_body,
        grid=grid,
        in_specs=[
            pl.BlockSpec((BT, IN_FEATURES), lambda i: (i, 0)),
            pl.BlockSpec((LANES, IN_FEATURES), lambda i: (0, 0)),
            pl.BlockSpec((NUM_EXPERTS, LANES), lambda i: (0, 0)),
        ],
        out_specs=pl.BlockSpec((NUM_EXPERTS, BT), lambda i: (0, i)),
        out_shape=jax.ShapeDtypeStruct((NUM_EXPERTS, T), jnp.float32),
    )(x, rwt, rb)

    wts = _sc_sparsemax(zt)

    out = pl.pallas_call(
        _dense_body,
        grid=grid,
        in_specs=[
            pl.BlockSpec((BT, IN_FEATURES), lambda i: (i, 0)),
            pl.BlockSpec((NCAT, IN_FEATURES), lambda i: (0, 0)),
            pl.BlockSpec((1, OUT_FEATURES), lambda i: (0, 0)),
            pl.BlockSpec((NUM_EXPERTS, BT), lambda i: (0, i)),
            pl.BlockSpec((NUM_EXPERTS * LORA_RANK, OUT_FEATURES),
                         lambda i: (0, 0)),
            pl.BlockSpec((NUM_EXPERTS, LANES), lambda i: (0, 0)),
        ],
        out_specs=pl.BlockSpec((BT, OUT_FEATURES), lambda i: (i, 0)),
        out_shape=jax.ShapeDtypeStruct((T, OUT_FEATURES), jnp.float32),
    )(x, wcat, b2, wts, bcat, rep)
    return out


# speedup vs baseline: 1.1605x; 1.1605x over previous
import functools

import jax
import jax.numpy as jnp
import numpy as np
from jax.experimental import pallas as pl

IN_FEATURES = 2048
OUT_FEATURES = 2048
NUM_EXPERTS = 8
LORA_RANK = 16
SPARSEGEN_LAMBDA = 0.5
LORA_SCALING = 2.0

LANES = 128
NCAT = OUT_FEATURES + 2 * LANES
BT = 512


def _body(x_ref, wcat_ref, b_ref, rb_ref, bcat_ref, rep_ref, o_ref):
    xb = x_ref[...].astype(jnp.bfloat16)

    big = jax.lax.dot_general(xb, wcat_ref[...], (((1,), (1,)), ((), ())),
                              preferred_element_type=jnp.float32)
    base = big[:, :OUT_FEATURES]
    h = big[:, OUT_FEATURES:OUT_FEATURES + LANES]
    z = big[:, OUT_FEATURES + LANES:]

    zt = z[:, :NUM_EXPERTS].T
    zt = (zt + rb_ref[...][:, :1]) * (1.0 / (1.0 - SPARSEGEN_LAMBDA))
    active = jnp.ones(zt.shape, jnp.float32)
    tau = jnp.zeros((1, zt.shape[1]), jnp.float32)
    for _ in range(NUM_EXPERTS):
        n = jnp.sum(active, axis=0, keepdims=True)
        s = jnp.sum(zt * active, axis=0, keepdims=True)
        tau = (s - 1.0) / n
        active = jnp.where(zt > tau, 1.0, 0.0)
    w = jnp.maximum(zt - tau, 0.0).T

    wrep = jnp.dot(w, rep_ref[...], preferred_element_type=jnp.float32)
    eo = jnp.dot((wrep * h).astype(jnp.bfloat16), bcat_ref[...],
                 preferred_element_type=jnp.float32)

    o_ref[...] = base + b_ref[...] + LORA_SCALING * eo


@functools.partial(jax.jit, static_argnames=())
def kernel(x, W_base, b_base, router_W, router_b, lora_A, lora_B):
    T = x.shape[0]
    b2 = b_base.reshape(1, OUT_FEATURES)
    rwt = jnp.zeros((LANES, IN_FEATURES), jnp.float32).at[:NUM_EXPERTS].set(
        router_W.T)
    rb = jnp.broadcast_to(router_b.reshape(NUM_EXPERTS, 1),
                          (NUM_EXPERTS, LANES))
    acat_t = lora_A.transpose(0, 2, 1).reshape(NUM_EXPERTS * LORA_RANK,
                                               IN_FEATURES)
    wcat = jnp.concatenate([W_base, acat_t, rwt], axis=0).astype(jnp.bfloat16)
    bcat = lora_B.reshape(NUM_EXPERTS * LORA_RANK,
                          OUT_FEATURES).astype(jnp.bfloat16)
    rep = np.zeros((NUM_EXPERTS, LANES), np.float32)
    for e in range(NUM_EXPERTS):
        rep[e, e * LORA_RANK:(e + 1) * LORA_RANK] = 1.0
    rep = jnp.asarray(rep)

    grid = (T // BT,)
    out = pl.pallas_call(
        _body,
        grid=grid,
        in_specs=[
            pl.BlockSpec((BT, IN_FEATURES), lambda i: (i, 0)),
            pl.BlockSpec((NCAT, IN_FEATURES), lambda i: (0, 0)),
            pl.BlockSpec((1, OUT_FEATURES), lambda i: (0, 0)),
            pl.BlockSpec((NUM_EXPERTS, LANES), lambda i: (0, 0)),
            pl.BlockSpec((NUM_EXPERTS * LORA_RANK, OUT_FEATURES),
                         lambda i: (0, 0)),
            pl.BlockSpec((NUM_EXPERTS, LANES), lambda i: (0, 0)),
        ],
        out_specs=pl.BlockSpec((BT, OUT_FEATURES), lambda i: (i, 0)),
        out_shape=jax.ShapeDtypeStruct((T, OUT_FEATURES), jnp.float32),
    )(x, wcat, b2, rb, bcat, rep)
    return out


# speedup vs baseline: 1.1782x; 1.0153x over previous
import functools

import jax
import jax.numpy as jnp
import numpy as np
from jax.experimental import pallas as pl

IN_FEATURES = 2048
OUT_FEATURES = 2048
NUM_EXPERTS = 8
LORA_RANK = 16
SPARSEGEN_LAMBDA = 0.5
LORA_SCALING = 2.0

LANES = 128
NCAT = OUT_FEATURES + 2 * LANES
BT = 1024


def _body(x_ref, wcat_ref, b_ref, rb_ref, bcat_ref, rep_ref, o_ref):
    xb = x_ref[...].astype(jnp.bfloat16)

    big = jax.lax.dot_general(xb, wcat_ref[...], (((1,), (1,)), ((), ())),
                              preferred_element_type=jnp.float32)
    base = big[:, :OUT_FEATURES]
    h = big[:, OUT_FEATURES:OUT_FEATURES + LANES]
    z = big[:, OUT_FEATURES + LANES:]

    zt = z[:, :NUM_EXPERTS].T
    zt = (zt + rb_ref[...][:, :1]) * (1.0 / (1.0 - SPARSEGEN_LAMBDA))
    active = jnp.ones(zt.shape, jnp.float32)
    tau = jnp.zeros((1, zt.shape[1]), jnp.float32)
    for _ in range(NUM_EXPERTS):
        n = jnp.sum(active, axis=0, keepdims=True)
        s = jnp.sum(zt * active, axis=0, keepdims=True)
        tau = (s - 1.0) / n
        active = jnp.where(zt > tau, 1.0, 0.0)
    w = jnp.maximum(zt - tau, 0.0).T

    wrep = jnp.dot(w, rep_ref[...], preferred_element_type=jnp.float32)
    eo = jnp.dot((wrep * h).astype(jnp.bfloat16), bcat_ref[...],
                 preferred_element_type=jnp.float32)

    o_ref[...] = base + b_ref[...] + LORA_SCALING * eo


@functools.partial(jax.jit, static_argnames=())
def kernel(x, W_base, b_base, router_W, router_b, lora_A, lora_B):
    T = x.shape[0]
    b2 = b_base.reshape(1, OUT_FEATURES)
    rwt = jnp.zeros((LANES, IN_FEATURES), jnp.float32).at[:NUM_EXPERTS].set(
        router_W.T)
    rb = jnp.broadcast_to(router_b.reshape(NUM_EXPERTS, 1),
                          (NUM_EXPERTS, LANES))
    acat_t = lora_A.transpose(0, 2, 1).reshape(NUM_EXPERTS * LORA_RANK,
                                               IN_FEATURES)
    wcat = jnp.concatenate([W_base, acat_t, rwt], axis=0).astype(jnp.bfloat16)
    bcat = lora_B.reshape(NUM_EXPERTS * LORA_RANK,
                          OUT_FEATURES).astype(jnp.bfloat16)
    rep = np.zeros((NUM_EXPERTS, LANES), np.float32)
    for e in range(NUM_EXPERTS):
        rep[e, e * LORA_RANK:(e + 1) * LORA_RANK] = 1.0
    rep = jnp.asarray(rep)

    grid = (T // BT,)
    out = pl.pallas_call(
        _body,
        grid=grid,
        in_specs=[
            pl.BlockSpec((BT, IN_FEATURES), lambda i: (i, 0)),
            pl.BlockSpec((NCAT, IN_FEATURES), lambda i: (0, 0)),
            pl.BlockSpec((1, OUT_FEATURES), lambda i: (0, 0)),
            pl.BlockSpec((NUM_EXPERTS, LANES), lambda i: (0, 0)),
            pl.BlockSpec((NUM_EXPERTS * LORA_RANK, OUT_FEATURES),
                         lambda i: (0, 0)),
            pl.BlockSpec((NUM_EXPERTS, LANES), lambda i: (0, 0)),
        ],
        out_specs=pl.BlockSpec((BT, OUT_FEATURES), lambda i: (i, 0)),
        out_shape=jax.ShapeDtypeStruct((T, OUT_FEATURES), jnp.float32),
    )(x, wcat, b2, rb, bcat, rep)
    return out
